# baseline (speedup 1.0000x reference)
import functools

import jax
import jax.numpy as jnp
from jax import lax
from jax.experimental import pallas as pl
from jax.experimental.pallas import tpu as pltpu
from jax.experimental.pallas import tpu_sc as plsc

R = 8
N = 524288
F = 2
B = 262144
MASK = N - 1
PRIME_I32 = -1640531535
RES = [16, 32, 64, 128, 256, 512, 1024, 2048]

NC = 2
NS = 16
NW = NC * NS
PTS_PER_W = B // NW
K = 256
NCHUNK = PTS_PER_W // K
NG = K // 16
M = R * 4 * K


def _sc_features():
    mesh = plsc.VectorSubcoreMesh(
        core_axis_name="c", subcore_axis_name="s", num_cores=NC,
        num_subcores=NS)

    @functools.partial(
        pl.kernel,
        out_type=jax.ShapeDtypeStruct((2 * R, B), jnp.float32),
        mesh=mesh,
        scratch_types=[
            pltpu.VMEM((K,), jnp.float32),
            pltpu.VMEM((K,), jnp.float32),
            pltpu.VMEM((M // 128, 128), jnp.int32),
            pltpu.VMEM((M, F), jnp.float32),
            pltpu.VMEM((2 * R, K), jnp.float32),
            pltpu.SemaphoreType.DMA,
        ],
        compiler_params=pltpu.CompilerParams(
            needs_layout_passes=False, use_tc_tiling_on_sc=False),
    )
    def feats_kernel(xt_hbm, table_hbm, ft_hbm, xb, yb, idxb, rows, fb, sem):
        cid = lax.axis_index("c")
        sid = lax.axis_index("s")
        wid = sid * NC + cid
        base0 = wid * PTS_PER_W
        iota = lax.iota(jnp.int32, 16)

        def chunk_body(t, _):
            base = base0 + t * K
            pltpu.sync_copy(xt_hbm.at[0, pl.ds(base, K)], xb)
            pltpu.sync_copy(xt_hbm.at[1, pl.ds(base, K)], yb)

            def idx_body(g, _):
                xv = xb[pl.ds(g * 16, 16)]
                yv = yb[pl.ds(g * 16, 16)]
                for r in range(R):
                    xs = xv * float(RES[r])
                    ys = yv * float(RES[r])
                    xi = xs.astype(jnp.int32)
                    yi = ys.astype(jnp.int32)
                    hy0 = yi * PRIME_I32
                    hy1 = (yi + 1) * PRIME_I32
                    xi1 = xi + 1
                    off = r * N
                    i00 = ((xi ^ hy0) & MASK) + off
                    i01 = ((xi ^ hy1) & MASK) + off
                    i10 = ((xi1 ^ hy0) & MASK) + off
                    i11 = ((xi1 ^ hy1) & MASK) + off
                    grow = g // 8
                    gcol = (g % 8) * 16
                    kr = K // 128
                    idxb[4 * r * kr + grow, pl.ds(gcol, 16)] = i00
                    idxb[(4 * r + 1) * kr + grow, pl.ds(gcol, 16)] = i01
                    idxb[(4 * r + 2) * kr + grow, pl.ds(gcol, 16)] = i10
                    idxb[(4 * r + 3) * kr + grow, pl.ds(gcol, 16)] = i11
                return 0

            lax.fori_loop(0, NG, idx_body, 0)

            def fire(j, _):
                pltpu.async_copy(
                    table_hbm.at[idxb.at[j]],
                    rows.at[pl.ds(j * 128, 128)], sem)
                return 0

            lax.fori_loop(0, M // 128, fire, 0)

            def drain(j, _):
                pltpu.make_async_copy(
                    table_hbm.at[idxb.at[0]],
                    rows.at[pl.ds(0, 128)], sem).wait()
                return 0

            lax.fori_loop(0, M // 128, drain, 0)

            def lerp_body(g, _):
                xv = xb[pl.ds(g * 16, 16)]
                yv = yb[pl.ds(g * 16, 16)]
                for r in range(R):
                    xs = xv * float(RES[r])
                    ys = yv * float(RES[r])
                    xi = xs.astype(jnp.int32)
                    yi = ys.astype(jnp.int32)
                    wx = xs - xi.astype(jnp.float32)
                    wy = ys - yi.astype(jnp.float32)
                    ux = 1.0 - wx
                    uy = 1.0 - wy
                    w00 = ux * uy
                    w01 = ux * wy
                    w10 = wx * uy
                    w11 = wx * wy
                    sb = 4 * r * K + g * 16
                    r00 = sb + iota
                    r01 = r00 + K
                    r10 = r00 + 2 * K
                    r11 = r00 + 3 * K
                    zero = jnp.zeros((16,), jnp.int32)
                    one = zero + 1
                    c00a = plsc.load_gather(rows, [r00, zero])
                    c01a = plsc.load_gather(rows, [r01, zero])
                    c10a = plsc.load_gather(rows, [r10, zero])
                    c11a = plsc.load_gather(rows, [r11, zero])
                    c00b = plsc.load_gather(rows, [r00, one])
                    c01b = plsc.load_gather(rows, [r01, one])
                    c10b = plsc.load_gather(rows, [r10, one])
                    c11b = plsc.load_gather(rows, [r11, one])
                    f0 = c00a * w00 + c01a * w01 + c10a * w10 + c11a * w11
                    f1 = c00b * w00 + c01b * w01 + c10b * w10 + c11b * w11
                    fb[2 * r, pl.ds(g * 16, 16)] = f0
                    fb[2 * r + 1, pl.ds(g * 16, 16)] = f1
                return 0

            lax.fori_loop(0, NG, lerp_body, 0)
            for j in range(2 * R):
                pltpu.sync_copy(fb.at[j], ft_hbm.at[j, pl.ds(base, K)])
            return 0

        lax.fori_loop(0, NCHUNK, chunk_body, 0)

    return feats_kernel


_FEATS = _sc_features()


def _mlp_body(f_ref, w1_ref, b1_ref, w2_ref, b2_ref, w3_ref, b3_ref, o_ref):
    f = f_ref[...]
    h = jnp.dot(w1_ref[...], f, preferred_element_type=jnp.float32)
    h = h + b1_ref[...]
    h = jnp.where(h >= 0.0, h, 0.01 * h)
    h = jnp.dot(w2_ref[...], h, preferred_element_type=jnp.float32) + b2_ref[...]
    h = jnp.where(h >= 0.0, h, 0.01 * h)
    o_ref[...] = jnp.dot(w3_ref[...], h, preferred_element_type=jnp.float32) + b3_ref[...]


def kernel(x, hash_features, W1, b1, W2, b2, W3, b3):
    xt = x.T
    table = hash_features.reshape(R * N, F)
    feats_t = _FEATS(xt, table)

    w3p = jnp.zeros((8, 16), jnp.float32).at[:3].set(W3)
    b3p = jnp.zeros((8, 1), jnp.float32).at[:3, 0].set(b3)
    bk = 2048
    out8 = pl.pallas_call(
        _mlp_body,
        grid=(B // bk,),
        in_specs=[
            pl.BlockSpec((2 * R, bk), lambda i: (0, i)),
            pl.BlockSpec((16, 16), lambda i: (0, 0)),
            pl.BlockSpec((16, 1), lambda i: (0, 0)),
            pl.BlockSpec((16, 16), lambda i: (0, 0)),
            pl.BlockSpec((16, 1), lambda i: (0, 0)),
            pl.BlockSpec((8, 16), lambda i: (0, 0)),
            pl.BlockSpec((8, 1), lambda i: (0, 0)),
        ],
        out_specs=pl.BlockSpec((8, bk), lambda i: (0, i)),
        out_shape=jax.ShapeDtypeStruct((8, B), jnp.float32),
    )(feats_t, W1, b1.reshape(16, 1), W2, b2.reshape(16, 1), w3p, b3p)
    return out8[:3].T


# speedup vs baseline: 1.0417x; 1.0417x over previous
import functools

import jax
import jax.numpy as jnp
from jax import lax
from jax.experimental import pallas as pl
from jax.experimental.pallas import tpu as pltpu
from jax.experimental.pallas import tpu_sc as plsc

R = 8
N = 524288
F = 2
B = 262144
MASK = N - 1
PRIME_I32 = -1640531535
RES = [16, 32, 64, 128, 256, 512, 1024, 2048]

NC = 2
NS = 16
NW = NC * NS
PTS_PER_W = B // NW
K = 256
NCHUNK = PTS_PER_W // K
NG = K // 16
M = R * 4 * K


def _sc_features():
    mesh = plsc.VectorSubcoreMesh(
        core_axis_name="c", subcore_axis_name="s", num_cores=NC,
        num_subcores=NS)

    @functools.partial(
        pl.kernel,
        out_type=jax.ShapeDtypeStruct((2 * R, B), jnp.float32),
        mesh=mesh,
        scratch_types=[
            pltpu.VMEM((2 * K,), jnp.float32),
            pltpu.VMEM((M // 128, 128), jnp.int32),
            pltpu.VMEM((M, F), jnp.float32),
            pltpu.VMEM((2 * R, K), jnp.float32),
            pltpu.SemaphoreType.DMA,
        ],
        compiler_params=pltpu.CompilerParams(
            needs_layout_passes=False, use_tc_tiling_on_sc=False),
    )
    def feats_kernel(x_hbm, table_hbm, ft_hbm, xyb, idxb, rows, fb, sem):
        cid = lax.axis_index("c")
        sid = lax.axis_index("s")
        wid = sid * NC + cid
        base0 = wid * PTS_PER_W
        iota = lax.iota(jnp.int32, 16)
        iota2 = iota * 2

        def chunk_body(t, _):
            base = base0 + t * K
            pltpu.sync_copy(x_hbm.at[pl.ds(2 * base, 2 * K)], xyb)

            def idx_body(g, _):
                pt2 = g * 32 + iota2
                xv = plsc.load_gather(xyb, [pt2])
                yv = plsc.load_gather(xyb, [pt2 + 1])
                for r in range(R):
                    xs = xv * float(RES[r])
                    ys = yv * float(RES[r])
                    xi = xs.astype(jnp.int32)
                    yi = ys.astype(jnp.int32)
                    hy0 = yi * PRIME_I32
                    hy1 = (yi + 1) * PRIME_I32
                    xi1 = xi + 1
                    off = r * N
                    i00 = ((xi ^ hy0) & MASK) + off
                    i01 = ((xi ^ hy1) & MASK) + off
                    i10 = ((xi1 ^ hy0) & MASK) + off
                    i11 = ((xi1 ^ hy1) & MASK) + off
                    grow = g // 8
                    gcol = (g % 8) * 16
                    kr = K // 128
                    idxb[4 * r * kr + grow, pl.ds(gcol, 16)] = i00
                    idxb[(4 * r + 1) * kr + grow, pl.ds(gcol, 16)] = i01
                    idxb[(4 * r + 2) * kr + grow, pl.ds(gcol, 16)] = i10
                    idxb[(4 * r + 3) * kr + grow, pl.ds(gcol, 16)] = i11
                return 0

            lax.fori_loop(0, NG, idx_body, 0)

            def fire(j, _):
                pltpu.async_copy(
                    table_hbm.at[idxb.at[j]],
                    rows.at[pl.ds(j * 128, 128)], sem)
                return 0

            lax.fori_loop(0, M // 128, fire, 0)

            def drain(j, _):
                pltpu.make_async_copy(
                    table_hbm.at[idxb.at[0]],
                    rows.at[pl.ds(0, 128)], sem).wait()
                return 0

            lax.fori_loop(0, M // 128, drain, 0)

            def lerp_body(g, _):
                pt2 = g * 32 + iota2
                xv = plsc.load_gather(xyb, [pt2])
                yv = plsc.load_gather(xyb, [pt2 + 1])
                for r in range(R):
                    xs = xv * float(RES[r])
                    ys = yv * float(RES[r])
                    xi = xs.astype(jnp.int32)
                    yi = ys.astype(jnp.int32)
                    wx = xs - xi.astype(jnp.float32)
                    wy = ys - yi.astype(jnp.float32)
                    ux = 1.0 - wx
                    uy = 1.0 - wy
                    w00 = ux * uy
                    w01 = ux * wy
                    w10 = wx * uy
                    w11 = wx * wy
                    sb = 4 * r * K + g * 16
                    r00 = sb + iota
                    r01 = r00 + K
                    r10 = r00 + 2 * K
                    r11 = r00 + 3 * K
                    zero = jnp.zeros((16,), jnp.int32)
                    one = zero + 1
                    c00a = plsc.load_gather(rows, [r00, zero])
                    c01a = plsc.load_gather(rows, [r01, zero])
                    c10a = plsc.load_gather(rows, [r10, zero])
                    c11a = plsc.load_gather(rows, [r11, zero])
                    c00b = plsc.load_gather(rows, [r00, one])
                    c01b = plsc.load_gather(rows, [r01, one])
                    c10b = plsc.load_gather(rows, [r10, one])
                    c11b = plsc.load_gather(rows, [r11, one])
                    f0 = c00a * w00 + c01a * w01 + c10a * w10 + c11a * w11
                    f1 = c00b * w00 + c01b * w01 + c10b * w10 + c11b * w11
                    fb[2 * r, pl.ds(g * 16, 16)] = f0
                    fb[2 * r + 1, pl.ds(g * 16, 16)] = f1
                return 0

            lax.fori_loop(0, NG, lerp_body, 0)
            for j in range(2 * R):
                pltpu.sync_copy(fb.at[j], ft_hbm.at[j, pl.ds(base, K)])
            return 0

        lax.fori_loop(0, NCHUNK, chunk_body, 0)

    return feats_kernel


_FEATS = _sc_features()


def _mlp_body(f_ref, w1_ref, b1_ref, w2_ref, b2_ref, w3_ref, b3_ref, o_ref):
    f = jnp.transpose(f_ref[...], (1, 0))
    h = lax.dot_general(f, w1_ref[...], (((1,), (1,)), ((), ())),
                        preferred_element_type=jnp.float32)
    h = h + b1_ref[...]
    h = jnp.where(h >= 0.0, h, 0.01 * h)
    h = lax.dot_general(h, w2_ref[...], (((1,), (1,)), ((), ())),
                        preferred_element_type=jnp.float32) + b2_ref[...]
    h = jnp.where(h >= 0.0, h, 0.01 * h)
    o_ref[...] = lax.dot_general(h, w3_ref[...], (((1,), (1,)), ((), ())),
                                 preferred_element_type=jnp.float32) + b3_ref[...]


def kernel(x, hash_features, W1, b1, W2, b2, W3, b3):
    table = hash_features.reshape(R * N, F)
    feats_t = _FEATS(x.reshape(2 * B), table)

    bk = 2048
    out = pl.pallas_call(
        _mlp_body,
        grid=(B // bk,),
        in_specs=[
            pl.BlockSpec((2 * R, bk), lambda i: (0, i)),
            pl.BlockSpec((16, 16), lambda i: (0, 0)),
            pl.BlockSpec((1, 16), lambda i: (0, 0)),
            pl.BlockSpec((16, 16), lambda i: (0, 0)),
            pl.BlockSpec((1, 16), lambda i: (0, 0)),
            pl.BlockSpec((3, 16), lambda i: (0, 0)),
            pl.BlockSpec((1, 3), lambda i: (0, 0)),
        ],
        out_specs=pl.BlockSpec((bk, 3), lambda i: (i, 0)),
        out_shape=jax.ShapeDtypeStruct((B, 3), jnp.float32),
    )(feats_t, W1, b1.reshape(1, 16), W2, b2.reshape(1, 16), W3,
      b3.reshape(1, 3))
    return out


# speedup vs baseline: 1.2131x; 1.1645x over previous
import functools

import jax
import jax.numpy as jnp
from jax import lax
from jax.experimental import pallas as pl
from jax.experimental.pallas import tpu as pltpu
from jax.experimental.pallas import tpu_sc as plsc

R = 8
N = 524288
F = 2
B = 262144
MASK = N - 1
PRIME_I32 = -1640531535
RES = [16, 32, 64, 128, 256, 512, 1024, 2048]

NC = 2
NS = 16
NW = NC * NS
PTS_PER_W = B // NW
K = 128
NCHUNK = PTS_PER_W // K
NG = K // 16
M = R * 4 * K


def _sc_features():
    mesh = plsc.VectorSubcoreMesh(
        core_axis_name="c", subcore_axis_name="s", num_cores=NC,
        num_subcores=NS)

    @functools.partial(
        pl.kernel,
        out_type=jax.ShapeDtypeStruct((2 * R, B), jnp.float32),
        mesh=mesh,
        scratch_types=[
            pltpu.VMEM((2 * K,), jnp.float32),
            pltpu.VMEM((M // 128, 128), jnp.int32),
            pltpu.VMEM((M, 16), jnp.float32),
            pltpu.VMEM((2 * R, K), jnp.float32),
            pltpu.SemaphoreType.DMA,
        ],
        compiler_params=pltpu.CompilerParams(
            needs_layout_passes=False, use_tc_tiling_on_sc=False),
    )
    def feats_kernel(x_hbm, table_hbm, ft_hbm, xyb, idxb, rows, fb, sem):
        cid = lax.axis_index("c")
        sid = lax.axis_index("s")
        wid = sid * NC + cid
        base0 = wid * PTS_PER_W
        iota = lax.iota(jnp.int32, 16)
        iota2 = iota * 2

        def chunk_body(t, _):
            base = base0 + t * K
            pltpu.sync_copy(x_hbm.at[pl.ds(2 * base, 2 * K)], xyb)

            def idx_body(g, _):
                pt2 = g * 32 + iota2
                xv = plsc.load_gather(xyb, [pt2])
                yv = plsc.load_gather(xyb, [pt2 + 1])
                for r in range(R):
                    xs = xv * float(RES[r])
                    ys = yv * float(RES[r])
                    xi = xs.astype(jnp.int32)
                    yi = ys.astype(jnp.int32)
                    hy0 = yi * PRIME_I32
                    hy1 = (yi + 1) * PRIME_I32
                    xi1 = xi + 1
                    off = r * (N // 8)
                    i00 = (((xi ^ hy0) & MASK) >> 3) + off
                    i01 = (((xi ^ hy1) & MASK) >> 3) + off
                    i10 = (((xi1 ^ hy0) & MASK) >> 3) + off
                    i11 = (((xi1 ^ hy1) & MASK) >> 3) + off
                    grow = g // 8
                    gcol = (g % 8) * 16
                    kr = K // 128
                    idxb[4 * r * kr + grow, pl.ds(gcol, 16)] = i00
                    idxb[(4 * r + 1) * kr + grow, pl.ds(gcol, 16)] = i01
                    idxb[(4 * r + 2) * kr + grow, pl.ds(gcol, 16)] = i10
                    idxb[(4 * r + 3) * kr + grow, pl.ds(gcol, 16)] = i11
                return 0

            lax.fori_loop(0, NG, idx_body, 0)

            def fire(j, _):
                pltpu.async_copy(
                    table_hbm.at[idxb.at[j]],
                    rows.at[pl.ds(j * 128, 128)], sem)
                return 0

            lax.fori_loop(0, M // 128, fire, 0)

            def drain(j, _):
                pltpu.make_async_copy(
                    table_hbm.at[idxb.at[0]],
                    rows.at[pl.ds(0, 128)], sem).wait()
                return 0

            lax.fori_loop(0, M // 128, drain, 0)

            def lerp_body(g, _):
                pt2 = g * 32 + iota2
                xv = plsc.load_gather(xyb, [pt2])
                yv = plsc.load_gather(xyb, [pt2 + 1])
                for r in range(R):
                    xs = xv * float(RES[r])
                    ys = yv * float(RES[r])
                    xi = xs.astype(jnp.int32)
                    yi = ys.astype(jnp.int32)
                    wx = xs - xi.astype(jnp.float32)
                    wy = ys - yi.astype(jnp.float32)
                    ux = 1.0 - wx
                    uy = 1.0 - wy
                    w00 = ux * uy
                    w01 = ux * wy
                    w10 = wx * uy
                    w11 = wx * wy
                    hy0 = yi * PRIME_I32
                    hy1 = (yi + 1) * PRIME_I32
                    xi1 = xi + 1
                    o00 = ((xi ^ hy0) & 7) * 2
                    o01 = ((xi ^ hy1) & 7) * 2
                    o10 = ((xi1 ^ hy0) & 7) * 2
                    o11 = ((xi1 ^ hy1) & 7) * 2
                    sb = 4 * r * K + g * 16
                    r00 = sb + iota
                    r01 = r00 + K
                    r10 = r00 + 2 * K
                    r11 = r00 + 3 * K
                    c00a = plsc.load_gather(rows, [r00, o00])
                    c01a = plsc.load_gather(rows, [r01, o01])
                    c10a = plsc.load_gather(rows, [r10, o10])
                    c11a = plsc.load_gather(rows, [r11, o11])
                    c00b = plsc.load_gather(rows, [r00, o00 + 1])
                    c01b = plsc.load_gather(rows, [r01, o01 + 1])
                    c10b = plsc.load_gather(rows, [r10, o10 + 1])
                    c11b = plsc.load_gather(rows, [r11, o11 + 1])
                    f0 = c00a * w00 + c01a * w01 + c10a * w10 + c11a * w11
                    f1 = c00b * w00 + c01b * w01 + c10b * w10 + c11b * w11
                    fb[2 * r, pl.ds(g * 16, 16)] = f0
                    fb[2 * r + 1, pl.ds(g * 16, 16)] = f1
                return 0

            lax.fori_loop(0, NG, lerp_body, 0)
            for j in range(2 * R):
                pltpu.sync_copy(fb.at[j], ft_hbm.at[j, pl.ds(base, K)])
            return 0

        lax.fori_loop(0, NCHUNK, chunk_body, 0)

    return feats_kernel


_FEATS = _sc_features()


def _mlp_body(f_ref, w1_ref, b1_ref, w2_ref, b2_ref, w3_ref, b3_ref, o_ref):
    f = jnp.transpose(f_ref[...], (1, 0))
    h = lax.dot_general(f, w1_ref[...], (((1,), (1,)), ((), ())),
                        preferred_element_type=jnp.float32)
    h = h + b1_ref[...]
    h = jnp.where(h >= 0.0, h, 0.01 * h)
    h = lax.dot_general(h, w2_ref[...], (((1,), (1,)), ((), ())),
                        preferred_element_type=jnp.float32) + b2_ref[...]
    h = jnp.where(h >= 0.0, h, 0.01 * h)
    o_ref[...] = lax.dot_general(h, w3_ref[...], (((1,), (1,)), ((), ())),
                                 preferred_element_type=jnp.float32) + b3_ref[...]


def kernel(x, hash_features, W1, b1, W2, b2, W3, b3):
    table = hash_features.reshape(R * N * F // 16, 16)
    feats_t = _FEATS(x.reshape(2 * B), table)

    bk = 2048
    out = pl.pallas_call(
        _mlp_body,
        grid=(B // bk,),
        in_specs=[
            pl.BlockSpec((2 * R, bk), lambda i: (0, i)),
            pl.BlockSpec((16, 16), lambda i: (0, 0)),
            pl.BlockSpec((1, 16), lambda i: (0, 0)),
            pl.BlockSpec((16, 16), lambda i: (0, 0)),
            pl.BlockSpec((1, 16), lambda i: (0, 0)),
            pl.BlockSpec((3, 16), lambda i: (0, 0)),
            pl.BlockSpec((1, 3), lambda i: (0, 0)),
        ],
        out_specs=pl.BlockSpec((bk, 3), lambda i: (i, 0)),
        out_shape=jax.ShapeDtypeStruct((B, 3), jnp.float32),
    )(feats_t, W1, b1.reshape(1, 16), W2, b2.reshape(1, 16), W3,
      b3.reshape(1, 3))
    return out


# speedup vs baseline: 6.1438x; 5.0647x over previous
import functools

import jax
import jax.numpy as jnp
from jax import lax
from jax.experimental import pallas as pl
from jax.experimental.pallas import tpu as pltpu
from jax.experimental.pallas import tpu_sc as plsc

R = 8
N = 524288
F = 2
B = 262144
MASK = N - 1
PRIME_I32 = -1640531535
RES = [16, 32, 64, 128, 256, 512, 1024, 2048]

NC = 2
NS = 16
NW = NC * NS
PTS_PER_W = B // NW
K = 512
NCHUNK = PTS_PER_W // K
NG = K // 16
M = R * 4 * K


def _planar_addr(h, r):
    return r * (N * F) + ((h >> 7) << 8) + (h & 127)


def _sc_features():
    mesh = plsc.VectorSubcoreMesh(
        core_axis_name="c", subcore_axis_name="s", num_cores=NC,
        num_subcores=NS)

    @functools.partial(
        pl.kernel,
        out_type=jax.ShapeDtypeStruct((2 * R, B), jnp.float32),
        mesh=mesh,
        scratch_types=[
            pltpu.VMEM((2 * K,), jnp.float32),
            pltpu.VMEM((2 * M // 128, 128), jnp.int32),
            pltpu.VMEM((2 * M,), jnp.float32),
            pltpu.VMEM((2 * R, K), jnp.float32),
            pltpu.SemaphoreType.DMA,
        ],
        compiler_params=pltpu.CompilerParams(
            needs_layout_passes=False, use_tc_tiling_on_sc=False),
    )
    def feats_kernel(x_hbm, table_hbm, ft_hbm, xyb, idxb, rows, fb, sem):
        cid = lax.axis_index("c")
        sid = lax.axis_index("s")
        wid = sid * NC + cid
        base0 = wid * PTS_PER_W
        iota = lax.iota(jnp.int32, 16)

        def chunk_body(t, _):
            base = base0 + t * K
            pltpu.sync_copy(x_hbm.at[pl.ds(2 * base, 2 * K)], xyb)

            def idx_body(g, _):
                po = (g // 8) * 256 + (g % 8) * 16
                xv = xyb[pl.ds(po, 16)]
                yv = xyb[pl.ds(po + 128, 16)]
                grow = g // 8
                gcol = (g % 8) * 16
                kr = K // 128
                for r in range(R):
                    xs = xv * float(RES[r])
                    ys = yv * float(RES[r])
                    xi = xs.astype(jnp.int32)
                    yi = ys.astype(jnp.int32)
                    hy0 = yi * PRIME_I32
                    hy1 = (yi + 1) * PRIME_I32
                    xi1 = xi + 1
                    a00 = _planar_addr((xi ^ hy0) & MASK, r)
                    a01 = _planar_addr((xi ^ hy1) & MASK, r)
                    a10 = _planar_addr((xi1 ^ hy0) & MASK, r)
                    a11 = _planar_addr((xi1 ^ hy1) & MASK, r)
                    r0 = 4 * r * kr + grow
                    idxb[r0, pl.ds(gcol, 16)] = a00
                    idxb[r0 + kr, pl.ds(gcol, 16)] = a01
                    idxb[r0 + 2 * kr, pl.ds(gcol, 16)] = a10
                    idxb[r0 + 3 * kr, pl.ds(gcol, 16)] = a11
                    r1 = r0 + M // 128
                    idxb[r1, pl.ds(gcol, 16)] = a00 + 128
                    idxb[r1 + kr, pl.ds(gcol, 16)] = a01 + 128
                    idxb[r1 + 2 * kr, pl.ds(gcol, 16)] = a10 + 128
                    idxb[r1 + 3 * kr, pl.ds(gcol, 16)] = a11 + 128
                return 0

            lax.fori_loop(0, NG, idx_body, 0)

            def fire(j, _):
                pltpu.async_copy(
                    table_hbm.at[idxb.at[j]],
                    rows.at[pl.ds(j * 128, 128)], sem)
                return 0

            lax.fori_loop(0, 2 * M // 128, fire, 0)

            def drain(j, _):
                pltpu.make_async_copy(
                    table_hbm.at[idxb.at[0]],
                    rows.at[pl.ds(0, 128)], sem).wait()
                return 0

            lax.fori_loop(0, 2 * M // 128, drain, 0)

            def lerp_body(g, _):
                po = (g // 8) * 256 + (g % 8) * 16
                xv = xyb[pl.ds(po, 16)]
                yv = xyb[pl.ds(po + 128, 16)]
                for r in range(R):
                    xs = xv * float(RES[r])
                    ys = yv * float(RES[r])
                    xi = xs.astype(jnp.int32)
                    yi = ys.astype(jnp.int32)
                    wx = xs - xi.astype(jnp.float32)
                    wy = ys - yi.astype(jnp.float32)
                    ux = 1.0 - wx
                    uy = 1.0 - wy
                    w00 = ux * uy
                    w01 = ux * wy
                    w10 = wx * uy
                    w11 = wx * wy
                    sb = 4 * r * K + g * 16
                    c00a = rows[pl.ds(sb, 16)]
                    c01a = rows[pl.ds(sb + K, 16)]
                    c10a = rows[pl.ds(sb + 2 * K, 16)]
                    c11a = rows[pl.ds(sb + 3 * K, 16)]
                    c00b = rows[pl.ds(M + sb, 16)]
                    c01b = rows[pl.ds(M + sb + K, 16)]
                    c10b = rows[pl.ds(M + sb + 2 * K, 16)]
                    c11b = rows[pl.ds(M + sb + 3 * K, 16)]
                    f0 = c00a * w00 + c01a * w01 + c10a * w10 + c11a * w11
                    f1 = c00b * w00 + c01b * w01 + c10b * w10 + c11b * w11
                    fb[2 * r, pl.ds(g * 16, 16)] = f0
                    fb[2 * r + 1, pl.ds(g * 16, 16)] = f1
                return 0

            lax.fori_loop(0, NG, lerp_body, 0)
            for j in range(2 * R):
                pltpu.sync_copy(fb.at[j], ft_hbm.at[j, pl.ds(base, K)])
            return 0

        lax.fori_loop(0, NCHUNK, chunk_body, 0)

    return feats_kernel


_FEATS = _sc_features()


def _mlp_body(f_ref, w1_ref, b1_ref, w2_ref, b2_ref, w3_ref, b3_ref, o_ref):
    f = jnp.transpose(f_ref[...], (1, 0))
    h = lax.dot_general(f, w1_ref[...], (((1,), (1,)), ((), ())),
                        preferred_element_type=jnp.float32)
    h = h + b1_ref[...]
    h = jnp.where(h >= 0.0, h, 0.01 * h)
    h = lax.dot_general(h, w2_ref[...], (((1,), (1,)), ((), ())),
                        preferred_element_type=jnp.float32) + b2_ref[...]
    h = jnp.where(h >= 0.0, h, 0.01 * h)
    o_ref[...] = lax.dot_general(h, w3_ref[...], (((1,), (1,)), ((), ())),
                                 preferred_element_type=jnp.float32) + b3_ref[...]


def kernel(x, hash_features, W1, b1, W2, b2, W3, b3):
    table = hash_features.reshape(R, N // 128, 128, F).transpose(
        0, 1, 3, 2).reshape(R * N * F)
    xp = x.reshape(B // 128, 128, 2).transpose(0, 2, 1).reshape(2 * B)
    feats_t = _FEATS(xp, table)

    bk = 2048
    out = pl.pallas_call(
        _mlp_body,
        grid=(B // bk,),
        in_specs=[
            pl.BlockSpec((2 * R, bk), lambda i: (0, i)),
            pl.BlockSpec((16, 16), lambda i: (0, 0)),
            pl.BlockSpec((1, 16), lambda i: (0, 0)),
            pl.BlockSpec((16, 16), lambda i: (0, 0)),
            pl.BlockSpec((1, 16), lambda i: (0, 0)),
            pl.BlockSpec((3, 16), lambda i: (0, 0)),
            pl.BlockSpec((1, 3), lambda i: (0, 0)),
        ],
        out_specs=pl.BlockSpec((bk, 3), lambda i: (i, 0)),
        out_shape=jax.ShapeDtypeStruct((B, 3), jnp.float32),
    )(feats_t, W1, b1.reshape(1, 16), W2, b2.reshape(1, 16), W3,
      b3.reshape(1, 3))
    return out


# speedup vs baseline: 6.3769x; 1.0379x over previous
import functools

import jax
import jax.numpy as jnp
from jax import lax
from jax.experimental import pallas as pl
from jax.experimental.pallas import tpu as pltpu
from jax.experimental.pallas import tpu_sc as plsc

R = 8
N = 524288
F = 2
B = 262144
MASK = N - 1
PRIME_I32 = -1640531535
RES = [16, 32, 64, 128, 256, 512, 1024, 2048]

NC = 2
NS = 16
NW = NC * NS
PTS_PER_W = B // NW
K = 256
NCHUNK = PTS_PER_W // K
NG = K // 16
M = R * 4 * K
NR = 2 * M // 128


def _planar_addr(h, r):
    return r * (N * F) + ((h >> 7) << 8) + (h & 127)


def _sc_features():
    mesh = plsc.VectorSubcoreMesh(
        core_axis_name="c", subcore_axis_name="s", num_cores=NC,
        num_subcores=NS)

    @functools.partial(
        pl.kernel,
        out_type=jax.ShapeDtypeStruct((2 * R, B), jnp.float32),
        mesh=mesh,
        scratch_types=[
            pltpu.VMEM((2 * 2 * K,), jnp.float32),
            pltpu.VMEM((2 * 2 * M // 128, 128), jnp.int32),
            pltpu.VMEM((2 * 2 * M,), jnp.float32),
            pltpu.VMEM((2 * R, K), jnp.float32),
            pltpu.SemaphoreType.DMA((2,)),
        ],
        compiler_params=pltpu.CompilerParams(
            needs_layout_passes=False, use_tc_tiling_on_sc=False),
    )
    def feats_kernel(x_hbm, table_hbm, ft_hbm, xyb, idxb, rows, fb, sem):
        cid = lax.axis_index("c")
        sid = lax.axis_index("s")
        wid = sid * NC + cid
        base0 = wid * PTS_PER_W
        iota = lax.iota(jnp.int32, 16)

        def stage(t, p):
            base = base0 + t * K
            xo = p * 2 * K
            pltpu.sync_copy(x_hbm.at[pl.ds(2 * base, 2 * K)],
                            xyb.at[pl.ds(xo, 2 * K)])

            def idx_body(g, _):
                po = xo + (g // 8) * 256 + (g % 8) * 16
                xv = xyb[pl.ds(po, 16)]
                yv = xyb[pl.ds(po + 128, 16)]
                grow = p * NR + g // 8
                gcol = (g % 8) * 16
                kr = K // 128
                for r in range(R):
                    xs = xv * float(RES[r])
                    ys = yv * float(RES[r])
                    xi = xs.astype(jnp.int32)
                    yi = ys.astype(jnp.int32)
                    hy0 = yi * PRIME_I32
                    hy1 = (yi + 1) * PRIME_I32
                    xi1 = xi + 1
                    a00 = _planar_addr((xi ^ hy0) & MASK, r)
                    a01 = _planar_addr((xi ^ hy1) & MASK, r)
                    a10 = _planar_addr((xi1 ^ hy0) & MASK, r)
                    a11 = _planar_addr((xi1 ^ hy1) & MASK, r)
                    r0 = 4 * r * kr + grow
                    idxb[r0, pl.ds(gcol, 16)] = a00
                    idxb[r0 + kr, pl.ds(gcol, 16)] = a01
                    idxb[r0 + 2 * kr, pl.ds(gcol, 16)] = a10
                    idxb[r0 + 3 * kr, pl.ds(gcol, 16)] = a11
                    r1 = r0 + M // 128
                    idxb[r1, pl.ds(gcol, 16)] = a00 + 128
                    idxb[r1 + kr, pl.ds(gcol, 16)] = a01 + 128
                    idxb[r1 + 2 * kr, pl.ds(gcol, 16)] = a10 + 128
                    idxb[r1 + 3 * kr, pl.ds(gcol, 16)] = a11 + 128
                return 0

            lax.fori_loop(0, NG, idx_body, 0)

            def fire(j, _):
                pltpu.async_copy(
                    table_hbm.at[idxb.at[p * NR + j]],
                    rows.at[pl.ds(p * 2 * M + j * 128, 128)], sem.at[p])
                return 0

            lax.fori_loop(0, NR, fire, 0)

        def finish(t, p):
            base = base0 + t * K

            def drain(j, _):
                pltpu.make_async_copy(
                    table_hbm.at[idxb.at[0]],
                    rows.at[pl.ds(0, 128)], sem.at[p]).wait()
                return 0

            lax.fori_loop(0, NR, drain, 0)

            def lerp_body(g, _):
                po = p * 2 * K + (g // 8) * 256 + (g % 8) * 16
                xv = xyb[pl.ds(po, 16)]
                yv = xyb[pl.ds(po + 128, 16)]
                ro = p * 2 * M
                for r in range(R):
                    xs = xv * float(RES[r])
                    ys = yv * float(RES[r])
                    xi = xs.astype(jnp.int32)
                    yi = ys.astype(jnp.int32)
                    wx = xs - xi.astype(jnp.float32)
                    wy = ys - yi.astype(jnp.float32)
                    ux = 1.0 - wx
                    uy = 1.0 - wy
                    w00 = ux * uy
                    w01 = ux * wy
                    w10 = wx * uy
                    w11 = wx * wy
                    sb = ro + 4 * r * K + g * 16
                    c00a = rows[pl.ds(sb, 16)]
                    c01a = rows[pl.ds(sb + K, 16)]
                    c10a = rows[pl.ds(sb + 2 * K, 16)]
                    c11a = rows[pl.ds(sb + 3 * K, 16)]
                    c00b = rows[pl.ds(M + sb, 16)]
                    c01b = rows[pl.ds(M + sb + K, 16)]
                    c10b = rows[pl.ds(M + sb + 2 * K, 16)]
                    c11b = rows[pl.ds(M + sb + 3 * K, 16)]
                    f0 = c00a * w00 + c01a * w01 + c10a * w10 + c11a * w11
                    f1 = c00b * w00 + c01b * w01 + c10b * w10 + c11b * w11
                    fb[2 * r, pl.ds(g * 16, 16)] = f0
                    fb[2 * r + 1, pl.ds(g * 16, 16)] = f1
                return 0

            lax.fori_loop(0, NG, lerp_body, 0)
            for j in range(2 * R):
                pltpu.sync_copy(fb.at[j], ft_hbm.at[j, pl.ds(base, K)])

        stage(0, 0)

        def chunk_body(t, _):
            p = t % 2
            stage(t, p)
            finish(t - 1, 1 - p)
            return 0

        lax.fori_loop(1, NCHUNK, chunk_body, 0)
        finish(NCHUNK - 1, (NCHUNK - 1) % 2)

    return feats_kernel


_FEATS = _sc_features()


def _mlp_body(f_ref, w1_ref, b1_ref, w2_ref, b2_ref, w3_ref, b3_ref, o_ref):
    f = jnp.transpose(f_ref[...], (1, 0))
    h = lax.dot_general(f, w1_ref[...], (((1,), (1,)), ((), ())),
                        preferred_element_type=jnp.float32)
    h = h + b1_ref[...]
    h = jnp.where(h >= 0.0, h, 0.01 * h)
    h = lax.dot_general(h, w2_ref[...], (((1,), (1,)), ((), ())),
                        preferred_element_type=jnp.float32) + b2_ref[...]
    h = jnp.where(h >= 0.0, h, 0.01 * h)
    o_ref[...] = lax.dot_general(h, w3_ref[...], (((1,), (1,)), ((), ())),
                                 preferred_element_type=jnp.float32) + b3_ref[...]


def kernel(x, hash_features, W1, b1, W2, b2, W3, b3):
    table = hash_features.reshape(R, N // 128, 128, F).transpose(
        0, 1, 3, 2).reshape(R * N * F)
    xp = x.reshape(B // 128, 128, 2).transpose(0, 2, 1).reshape(2 * B)
    feats_t = _FEATS(xp, table)

    bk = 2048
    out = pl.pallas_call(
        _mlp_body,
        grid=(B // bk,),
        in_specs=[
            pl.BlockSpec((2 * R, bk), lambda i: (0, i)),
            pl.BlockSpec((16, 16), lambda i: (0, 0)),
            pl.BlockSpec((1, 16), lambda i: (0, 0)),
            pl.BlockSpec((16, 16), lambda i: (0, 0)),
            pl.BlockSpec((1, 16), lambda i: (0, 0)),
            pl.BlockSpec((3, 16), lambda i: (0, 0)),
            pl.BlockSpec((1, 3), lambda i: (0, 0)),
        ],
        out_specs=pl.BlockSpec((bk, 3), lambda i: (i, 0)),
        out_shape=jax.ShapeDtypeStruct((B, 3), jnp.float32),
    )(feats_t, W1, b1.reshape(1, 16), W2, b2.reshape(1, 16), W3,
      b3.reshape(1, 3))
    return out


# speedup vs baseline: 10.8619x; 1.7033x over previous
import functools

import jax
import jax.numpy as jnp
from jax import lax
from jax.experimental import pallas as pl
from jax.experimental.pallas import tpu as pltpu
from jax.experimental.pallas import tpu_sc as plsc

R = 8
N = 524288
F = 2
B = 262144
MASK = N - 1
PRIME_I32 = -1640531535
RES = [16, 32, 64, 128, 256, 512, 1024, 2048]

NC = 2
NS = 16
NW = NC * NS
PTS_PER_W = B // NW
K = 256
NCHUNK = PTS_PER_W // K
NG = K // 16
GL = 4
HL = R - GL
M = HL * 4 * K
NR = 2 * M // 128

GSIZE = [(RES[r] + 1) * (RES[r] + 1) for r in range(GL)]
GPAD = [-(-s // 128) * 128 for s in GSIZE]
GOFF = [2 * sum(GPAD[:r]) for r in range(GL)]
GROWS = [s // 128 for s in GPAD]
GBUF = 2 * sum(GPAD)
IDXROWS = max(2 * NR, 2 * max(GROWS))


def _planar_addr(h, r):
    return r * (N * F) + ((h >> 7) << 8) + (h & 127)


def _sc_features():
    mesh = plsc.VectorSubcoreMesh(
        core_axis_name="c", subcore_axis_name="s", num_cores=NC,
        num_subcores=NS)

    @functools.partial(
        pl.kernel,
        out_type=jax.ShapeDtypeStruct((2 * R, B), jnp.float32),
        mesh=mesh,
        scratch_types=[
            pltpu.VMEM((2 * 2 * K,), jnp.float32),
            pltpu.VMEM((IDXROWS, 128), jnp.int32),
            pltpu.VMEM((2 * 2 * M,), jnp.float32),
            pltpu.VMEM((GBUF,), jnp.float32),
            pltpu.VMEM((2 * R, K), jnp.float32),
            pltpu.SemaphoreType.DMA((2,)),
        ],
        compiler_params=pltpu.CompilerParams(
            needs_layout_passes=False, use_tc_tiling_on_sc=False),
    )
    def feats_kernel(x_hbm, table_hbm, ft_hbm, xyb, idxb, rows, gbuf, fb, sem):
        cid = lax.axis_index("c")
        sid = lax.axis_index("s")
        wid = sid * NC + cid
        base0 = wid * PTS_PER_W
        iota = lax.iota(jnp.int32, 16)

        for r in range(GL):
            res1 = RES[r] + 1
            smax = GSIZE[r] - 1

            def gfill(j, _, r=r, res1=res1, smax=smax):
                for q in range(8):
                    cell = jnp.minimum(j * 128 + q * 16 + iota, smax)
                    gx = cell // res1
                    gy = cell - gx * res1
                    h = (gx ^ (gy * PRIME_I32)) & MASK
                    a0 = _planar_addr(h, r)
                    idxb[j, pl.ds(q * 16, 16)] = a0
                    idxb[GROWS[r] + j, pl.ds(q * 16, 16)] = a0 + 128
                return 0

            lax.fori_loop(0, GROWS[r], gfill, 0)

            def gfire(j, _, r=r):
                pltpu.async_copy(
                    table_hbm.at[idxb.at[j]],
                    gbuf.at[pl.ds(GOFF[r] + j * 128, 128)], sem.at[0])
                pltpu.async_copy(
                    table_hbm.at[idxb.at[GROWS[r] + j]],
                    gbuf.at[pl.ds(GOFF[r] + GPAD[r] + j * 128, 128)],
                    sem.at[0])
                return 0

            lax.fori_loop(0, GROWS[r], gfire, 0)

            def gdrain(j, _):
                pltpu.make_async_copy(
                    table_hbm.at[idxb.at[0]],
                    gbuf.at[pl.ds(0, 128)], sem.at[0]).wait()
                return 0

            lax.fori_loop(0, 2 * GROWS[r], gdrain, 0)

        def stage(t, p):
            base = base0 + t * K
            xo = p * 2 * K
            pltpu.sync_copy(x_hbm.at[pl.ds(2 * base, 2 * K)],
                            xyb.at[pl.ds(xo, 2 * K)])

            def idx_body(g, _):
                po = xo + (g // 8) * 256 + (g % 8) * 16
                xv = xyb[pl.ds(po, 16)]
                yv = xyb[pl.ds(po + 128, 16)]
                grow = p * NR + g // 8
                gcol = (g % 8) * 16
                kr = K // 128
                for r in range(GL, R):
                    xs = xv * float(RES[r])
                    ys = yv * float(RES[r])
                    xi = xs.astype(jnp.int32)
                    yi = ys.astype(jnp.int32)
                    hy0 = yi * PRIME_I32
                    hy1 = (yi + 1) * PRIME_I32
                    xi1 = xi + 1
                    a00 = _planar_addr((xi ^ hy0) & MASK, r)
                    a01 = _planar_addr((xi ^ hy1) & MASK, r)
                    a10 = _planar_addr((xi1 ^ hy0) & MASK, r)
                    a11 = _planar_addr((xi1 ^ hy1) & MASK, r)
                    r0 = 4 * (r - GL) * kr + grow
                    idxb[r0, pl.ds(gcol, 16)] = a00
                    idxb[r0 + kr, pl.ds(gcol, 16)] = a01
                    idxb[r0 + 2 * kr, pl.ds(gcol, 16)] = a10
                    idxb[r0 + 3 * kr, pl.ds(gcol, 16)] = a11
                    r1 = r0 + M // 128
                    idxb[r1, pl.ds(gcol, 16)] = a00 + 128
                    idxb[r1 + kr, pl.ds(gcol, 16)] = a01 + 128
                    idxb[r1 + 2 * kr, pl.ds(gcol, 16)] = a10 + 128
                    idxb[r1 + 3 * kr, pl.ds(gcol, 16)] = a11 + 128
                return 0

            lax.fori_loop(0, NG, idx_body, 0)

            def fire(j, _):
                pltpu.async_copy(
                    table_hbm.at[idxb.at[p * NR + j]],
                    rows.at[pl.ds(p * 2 * M + j * 128, 128)], sem.at[p])
                return 0

            lax.fori_loop(0, NR, fire, 0)

        def finish(t, p):
            base = base0 + t * K

            def drain(j, _):
                pltpu.make_async_copy(
                    table_hbm.at[idxb.at[0]],
                    rows.at[pl.ds(0, 128)], sem.at[p]).wait()
                return 0

            lax.fori_loop(0, NR, drain, 0)

            def lerp_body(g, _):
                po = p * 2 * K + (g // 8) * 256 + (g % 8) * 16
                xv = xyb[pl.ds(po, 16)]
                yv = xyb[pl.ds(po + 128, 16)]
                ro = p * 2 * M
                for r in range(R):
                    xs = xv * float(RES[r])
                    ys = yv * float(RES[r])
                    xi = xs.astype(jnp.int32)
                    yi = ys.astype(jnp.int32)
                    wx = xs - xi.astype(jnp.float32)
                    wy = ys - yi.astype(jnp.float32)
                    ux = 1.0 - wx
                    uy = 1.0 - wy
                    w00 = ux * uy
                    w01 = ux * wy
                    w10 = wx * uy
                    w11 = wx * wy
                    if r < GL:
                        c00 = GOFF[r] + xi * (RES[r] + 1) + yi
                        c10 = c00 + (RES[r] + 1)
                        gp = GPAD[r]
                        c00a = plsc.load_gather(gbuf, [c00])
                        c01a = plsc.load_gather(gbuf, [c00 + 1])
                        c10a = plsc.load_gather(gbuf, [c10])
                        c11a = plsc.load_gather(gbuf, [c10 + 1])
                        c00b = plsc.load_gather(gbuf, [c00 + gp])
                        c01b = plsc.load_gather(gbuf, [c00 + gp + 1])
                        c10b = plsc.load_gather(gbuf, [c10 + gp])
                        c11b = plsc.load_gather(gbuf, [c10 + gp + 1])
                    else:
                        sb = ro + 4 * (r - GL) * K + g * 16
                        c00a = rows[pl.ds(sb, 16)]
                        c01a = rows[pl.ds(sb + K, 16)]
                        c10a = rows[pl.ds(sb + 2 * K, 16)]
                        c11a = rows[pl.ds(sb + 3 * K, 16)]
                        c00b = rows[pl.ds(M + sb, 16)]
                        c01b = rows[pl.ds(M + sb + K, 16)]
                        c10b = rows[pl.ds(M + sb + 2 * K, 16)]
                        c11b = rows[pl.ds(M + sb + 3 * K, 16)]
                    f0 = c00a * w00 + c01a * w01 + c10a * w10 + c11a * w11
                    f1 = c00b * w00 + c01b * w01 + c10b * w10 + c11b * w11
                    fb[2 * r, pl.ds(g * 16, 16)] = f0
                    fb[2 * r + 1, pl.ds(g * 16, 16)] = f1
                return 0

            lax.fori_loop(0, NG, lerp_body, 0)
            for j in range(2 * R):
                pltpu.sync_copy(fb.at[j], ft_hbm.at[j, pl.ds(base, K)])

        stage(0, 0)

        def chunk_body(t, _):
            p = t % 2
            stage(t, p)
            finish(t - 1, 1 - p)
            return 0

        lax.fori_loop(1, NCHUNK, chunk_body, 0)
        finish(NCHUNK - 1, (NCHUNK - 1) % 2)

    return feats_kernel


_FEATS = _sc_features()


def _mlp_body(f_ref, w1_ref, b1_ref, w2_ref, b2_ref, w3_ref, b3_ref, o_ref):
    f = jnp.transpose(f_ref[...], (1, 0))
    h = lax.dot_general(f, w1_ref[...], (((1,), (1,)), ((), ())),
                        preferred_element_type=jnp.float32)
    h = h + b1_ref[...]
    h = jnp.where(h >= 0.0, h, 0.01 * h)
    h = lax.dot_general(h, w2_ref[...], (((1,), (1,)), ((), ())),
                        preferred_element_type=jnp.float32) + b2_ref[...]
    h = jnp.where(h >= 0.0, h, 0.01 * h)
    o_ref[...] = lax.dot_general(h, w3_ref[...], (((1,), (1,)), ((), ())),
                                 preferred_element_type=jnp.float32) + b3_ref[...]


def kernel(x, hash_features, W1, b1, W2, b2, W3, b3):
    table = hash_features.reshape(R, N // 128, 128, F).transpose(
        0, 1, 3, 2).reshape(R * N * F)
    xp = x.reshape(B // 128, 128, 2).transpose(0, 2, 1).reshape(2 * B)
    feats_t = _FEATS(xp, table)

    bk = 2048
    out = pl.pallas_call(
        _mlp_body,
        grid=(B // bk,),
        in_specs=[
            pl.BlockSpec((2 * R, bk), lambda i: (0, i)),
            pl.BlockSpec((16, 16), lambda i: (0, 0)),
            pl.BlockSpec((1, 16), lambda i: (0, 0)),
            pl.BlockSpec((16, 16), lambda i: (0, 0)),
            pl.BlockSpec((1, 16), lambda i: (0, 0)),
            pl.BlockSpec((3, 16), lambda i: (0, 0)),
            pl.BlockSpec((1, 3), lambda i: (0, 0)),
        ],
        out_specs=pl.BlockSpec((bk, 3), lambda i: (i, 0)),
        out_shape=jax.ShapeDtypeStruct((B, 3), jnp.float32),
    )(feats_t, W1, b1.reshape(1, 16), W2, b2.reshape(1, 16), W3,
      b3.reshape(1, 3))
    return out
